# baseline (speedup 1.0000x reference)
import functools
import math

import jax
import jax.numpy as jnp
from jax import lax
from jax.experimental import pallas as pl
from jax.experimental.pallas import tpu as pltpu
from jax.experimental.pallas import tpu_sc as plsc

VOCAB = 1000000
D = 64
SCALE = math.sqrt(D)
NC, NS = 2, 16
NW = NC * NS
CH = 200


def _emb_body(B, BPW, NCHUNK,
              x_hbm, table_hbm, out_hbm,
              idx_all, n0, n1, n2, w0, w1,
              g0, g1, g2, o0, o1):
    nar = (n0, n1, n2)
    wide = (w0, w1)
    gsem = (g0, g1, g2)
    osem = (o0, o1)
    wid = lax.axis_index("s") * NC + lax.axis_index("c")
    base = wid * BPW
    x1base = wid * NCHUNK

    pltpu.sync_copy(x_hbm.at[pl.ds(base, BPW)], idx_all)

    def start_gather(c, b):
        pltpu.async_copy(table_hbm.at[idx_all.at[pl.ds(c * CH, CH)]],
                         nar[b], gsem[b])

    def wait_gather(c, b):
        pltpu.make_async_copy(table_hbm.at[idx_all.at[pl.ds(c * CH, CH)]],
                              nar[b], gsem[b]).wait()

    def scale(b, v):
        src, dst = nar[b], wide[v]

        @plsc.parallel_loop(0, CH, unroll=4)
        def _(i):
            for col in range(D // 16):
                sl = pl.ds(col * 16, 16)
                dst[i, sl] = src[i, sl]

    def start_out(c, v):
        pltpu.async_copy(wide[v], out_hbm.at[x1base + c], osem[v])

    def wait_out(c, v):
        pltpu.make_async_copy(wide[v], out_hbm.at[x1base + c],
                              osem[v]).wait()

    def body(c, head, tail, b, v):
        wait_gather(c, b)
        if not head:
            wait_out(c - 2, v)
        scale(b, v)
        start_out(c, v)
        if not tail:
            start_gather(c + 3, b)

    start_gather(0, 0)
    start_gather(1, 1)
    start_gather(2, 2)
    body(0, True, False, 0, 0)
    body(1, True, False, 1, 1)
    lo = 2
    hi = NCHUNK - 3
    n6, rem = divmod(hi - lo, 6)

    if n6 > 0:
        def outer(t, _):
            c0 = lo + t * 6
            for j in range(6):
                body(c0 + j, False, False, (lo + j) % 3, (lo + j) % 2)
            return 0
        lax.fori_loop(0, n6, outer, 0)
    for c in range(lo + n6 * 6, hi):
        body(c, False, False, c % 3, c % 2)
    for c in range(hi, NCHUNK):
        body(c, False, True, c % 3, c % 2)
    wait_out(NCHUNK - 2, (NCHUNK - 2) % 2)
    wait_out(NCHUNK - 1, (NCHUNK - 1) % 2)


@functools.partial(jax.jit, static_argnames=("B", "R"))
def _emb(xf, table, B, R):
    BPW = B // NW
    NCHUNK = BPW // CH
    body = functools.partial(_emb_body, B, BPW, NCHUNK)
    run = pl.kernel(
        body,
        out_type=jax.ShapeDtypeStruct((R, CH, 2 * D), jnp.float32),
        mesh=plsc.VectorSubcoreMesh(core_axis_name="c", subcore_axis_name="s",
                                    num_cores=NC, num_subcores=NS),
        compiler_params=pltpu.CompilerParams(use_tc_tiling_on_sc=False),
        scratch_types=[
            pltpu.VMEM((BPW,), jnp.int32),
            pltpu.VMEM((CH, D), jnp.float32),
            pltpu.VMEM((CH, D), jnp.float32),
            pltpu.VMEM((CH, D), jnp.float32),
            pltpu.VMEM((CH, 2 * D), jnp.float32),
            pltpu.VMEM((CH, 2 * D), jnp.float32),
            pltpu.SemaphoreType.DMA,
            pltpu.SemaphoreType.DMA,
            pltpu.SemaphoreType.DMA,
            pltpu.SemaphoreType.DMA,
            pltpu.SemaphoreType.DMA,
        ],
    )
    return run(xf, table)


def kernel(x, table):
    R, C = x.shape
    assert C == CH and table.shape == (VOCAB, D)
    B = R * C
    xf = x.reshape(B).astype(jnp.int32)
    out = _emb(xf, table * SCALE, B, R)
    return out[:, :, :D]
